# baseline (speedup 1.0000x reference)
import functools

import jax
import jax.numpy as jnp
from jax import lax
from jax.experimental import pallas as pl
from jax.experimental.pallas import tpu as pltpu
from jax.experimental.pallas import tpu_sc as plsc

VOCAB = 1000000
EMBED = 64
SEQ = 77
BATCH = 4096

NC = 2
NS = 16
NW = NC * NS

BBLK = BATCH // NW


def _make_kernel():
    mesh = plsc.VectorSubcoreMesh(core_axis_name="c", subcore_axis_name="s")

    @functools.partial(
        pl.kernel,
        mesh=mesh,
        out_type=jax.ShapeDtypeStruct((SEQ, EMBED, BATCH), jnp.float32),
        scratch_types=[
            pltpu.VMEM((EMBED, 128), jnp.float32),
            pltpu.VMEM((BBLK,), jnp.int32),
            pltpu.VMEM((BBLK,), jnp.int32),
            pltpu.VMEM((BBLK,), jnp.int32),
            pltpu.VMEM((BBLK, 128), jnp.float32),
            pltpu.VMEM((EMBED, BBLK), jnp.float32),
            pltpu.SemaphoreType.DMA,
        ],
        compiler_params=pltpu.CompilerParams(
            use_tc_tiling_on_sc=True, needs_layout_passes=False),
    )
    def body(ids_hbm, tab2_hbm, pos_hbm, out_hbm, pos_v, ids_v, hidx_v, par_v,
             gbuf, obuf, sem):
        wid = lax.axis_index("s") * NC + lax.axis_index("c")
        b0 = wid * BBLK

        pltpu.sync_copy(pos_hbm, pos_v)
        iota = lax.iota(jnp.int32, 16)

        def s_body(s, carry):
            pltpu.sync_copy(ids_hbm.at[s, pl.ds(b0, BBLK)], ids_v)
            for q in range(BBLK // 16):
                v = ids_v[pl.ds(q * 16, 16)]
                hidx_v[pl.ds(q * 16, 16)] = lax.shift_right_logical(v, 1)
                par_v[pl.ds(q * 16, 16)] = lax.bitwise_and(v, 1) * EMBED
            pltpu.async_copy(tab2_hbm.at[hidx_v], gbuf, sem).wait()

            s_splat = lax.broadcast(s, (16,))
            pos_q = [
                plsc.load_gather(pos_v, [q * 16 + iota, s_splat])
                for q in range(EMBED // 16)
            ]

            par_vecs = [par_v[pl.ds(jv * 16, 16)] for jv in range(BBLK // 16)]
            jvecs = [jv * 16 + iota for jv in range(BBLK // 16)]
            for d in range(EMBED):
                pq = pos_q[d // 16][d % 16]
                for jv in range(BBLK // 16):
                    vals = plsc.load_gather(gbuf, [jvecs[jv], par_vecs[jv] + d])
                    obuf[d, pl.ds(jv * 16, 16)] = vals + pq

            pltpu.sync_copy(obuf, out_hbm.at[s, :, pl.ds(b0, BBLK)])
            return carry

        lax.fori_loop(0, SEQ, s_body, 0)

    return body


_sc_kernel = _make_kernel()


def kernel(input_ids, token_embedding, position_embedding):
    ids_p = input_ids.T.astype(jnp.int32)
    tab2 = token_embedding.reshape(VOCAB // 2, 128)
    pos_p = position_embedding.T
    out_p = _sc_kernel(ids_p, tab2, pos_p)
    return out_p.transpose(2, 0, 1)


# speedup vs baseline: 1.1000x; 1.1000x over previous
import functools

import jax
import jax.numpy as jnp
from jax import lax
from jax.experimental import pallas as pl
from jax.experimental.pallas import tpu as pltpu
from jax.experimental.pallas import tpu_sc as plsc

VOCAB = 1000000
EMBED = 64
SEQ = 77
BATCH = 4096

NC = 2
NS = 16
NW = NC * NS

BBLK = BATCH // NW
NGB = 4
NOB = 2


def _make_kernel():
    mesh = plsc.VectorSubcoreMesh(core_axis_name="c", subcore_axis_name="s")

    @functools.partial(
        pl.kernel,
        mesh=mesh,
        out_type=jax.ShapeDtypeStruct((SEQ, EMBED, BATCH), jnp.float32),
        scratch_types=[
            pltpu.VMEM((EMBED, 128), jnp.float32),
            pltpu.VMEM((SEQ, BBLK), jnp.int32),
            pltpu.VMEM((SEQ, BBLK), jnp.int32),
            pltpu.VMEM((SEQ, BBLK), jnp.int32),
            pltpu.VMEM((NGB, BBLK, 128), jnp.float32),
            pltpu.VMEM((NOB, EMBED, BBLK), jnp.float32),
            pltpu.SemaphoreType.DMA((NGB,)),
            pltpu.SemaphoreType.DMA((NOB,)),
        ],
        compiler_params=pltpu.CompilerParams(
            use_tc_tiling_on_sc=True, needs_layout_passes=False),
    )
    def body(ids_hbm, tab2_hbm, pos_hbm, out_hbm, pos_v, ids_v, hidx_v, off_v,
             gbuf, obuf, gsem, wsem):
        wid = lax.axis_index("s") * NC + lax.axis_index("c")
        b0 = wid * BBLK

        pltpu.sync_copy(pos_hbm, pos_v)
        pltpu.sync_copy(ids_hbm.at[:, pl.ds(b0, BBLK)], ids_v)
        iota = lax.iota(jnp.int32, 16)

        def prep_body(s, carry):
            for q in range(BBLK // 16):
                v = ids_v[s, pl.ds(q * 16, 16)]
                hidx_v[s, pl.ds(q * 16, 16)] = lax.shift_right_logical(v, 1)
                off_v[s, pl.ds(q * 16, 16)] = lax.bitwise_and(v, 1) * EMBED
            return carry

        lax.fori_loop(0, SEQ, prep_body, 0)

        def gather_descr(s):
            slot = lax.rem(s, NGB)
            return pltpu.make_async_copy(
                tab2_hbm.at[hidx_v.at[s]], gbuf.at[slot], gsem.at[slot])

        def write_descr(s):
            slot = lax.rem(s, NOB)
            return pltpu.make_async_copy(
                obuf.at[slot], out_hbm.at[s, :, pl.ds(b0, BBLK)],
                wsem.at[slot])

        for s in range(NGB - 1):
            gather_descr(s).start()

        jvecs = [jv * 16 + iota for jv in range(BBLK // 16)]

        def s_body(s, carry):
            @pl.when(s + NGB - 1 < SEQ)
            def _():
                gather_descr(s + NGB - 1).start()

            gather_descr(s).wait()

            @pl.when(s >= NOB)
            def _():
                write_descr(s - NOB).wait()

            gslot = lax.broadcast(lax.rem(s, NGB), (16,))
            oslot = lax.rem(s, NOB)
            s_splat = lax.broadcast(s, (16,))
            pos_q = [
                plsc.load_gather(pos_v, [q * 16 + iota, s_splat])
                for q in range(EMBED // 16)
            ]
            off_vecs = [off_v[s, pl.ds(jv * 16, 16)] for jv in range(BBLK // 16)]
            ob = obuf.at[oslot]
            for d in range(EMBED):
                pq = pos_q[d // 16][d % 16]
                for jv in range(BBLK // 16):
                    vals = plsc.load_gather(
                        gbuf, [gslot, jvecs[jv], off_vecs[jv] + d])
                    ob[d, pl.ds(jv * 16, 16)] = vals + pq

            write_descr(s).start()
            return carry

        lax.fori_loop(0, SEQ, s_body, 0)
        write_descr(SEQ - 2).wait()
        write_descr(SEQ - 1).wait()

    return body


_sc_kernel = _make_kernel()


def kernel(input_ids, token_embedding, position_embedding):
    ids_p = input_ids.T.astype(jnp.int32)
    tab2 = token_embedding.reshape(VOCAB // 2, 128)
    pos_p = position_embedding.T
    out_p = _sc_kernel(ids_p, tab2, pos_p)
    return out_p.transpose(2, 0, 1)


# speedup vs baseline: 1.2887x; 1.1715x over previous
import functools

import jax
import jax.numpy as jnp
from jax import lax
from jax.experimental import pallas as pl
from jax.experimental.pallas import tpu as pltpu
from jax.experimental.pallas import tpu_sc as plsc

VOCAB = 1000000
EMBED = 64
SEQ = 77
BATCH = 4096

NC = 2
NS = 16
NW = NC * NS

BBLK = BATCH // NW
NGB = 4
NOB = 2


def _make_kernel():
    mesh = plsc.VectorSubcoreMesh(core_axis_name="c", subcore_axis_name="s")

    @functools.partial(
        pl.kernel,
        mesh=mesh,
        out_type=jax.ShapeDtypeStruct((SEQ, EMBED, BATCH), jnp.float32),
        scratch_types=[
            pltpu.VMEM((EMBED, 128), jnp.float32),
            pltpu.VMEM((SEQ, BBLK), jnp.int32),
            pltpu.VMEM((NGB, BBLK, 128), jnp.float32),
            pltpu.VMEM((NOB, EMBED, BBLK), jnp.float32),
            pltpu.SemaphoreType.DMA((NGB,)),
            pltpu.SemaphoreType.DMA((NOB,)),
        ],
        compiler_params=pltpu.CompilerParams(
            use_tc_tiling_on_sc=True, needs_layout_passes=False),
    )
    def body(ids_hbm, tab_hbm, pos_hbm, out_hbm, pos_v, ids_v, gbuf, obuf,
             gsem, wsem):
        wid = lax.axis_index("s") * NC + lax.axis_index("c")
        b0 = wid * BBLK

        pltpu.sync_copy(pos_hbm, pos_v)
        pltpu.sync_copy(ids_hbm.at[:, pl.ds(b0, BBLK)], ids_v)
        iota = lax.iota(jnp.int32, 16)

        def gather_descr(s):
            slot = lax.rem(s, NGB)
            return pltpu.make_async_copy(
                tab_hbm.at[ids_v.at[s]], gbuf.at[slot], gsem.at[slot])

        def write_descr(s):
            slot = lax.rem(s, NOB)
            return pltpu.make_async_copy(
                obuf.at[slot], out_hbm.at[s, :, pl.ds(b0, BBLK)],
                wsem.at[slot])

        for s in range(NGB - 1):
            gather_descr(s).start()

        jvecs = [jv * 16 + iota for jv in range(BBLK // 16)]

        def s_body(s, carry):
            @pl.when(s + NGB - 1 < SEQ)
            def _():
                gather_descr(s + NGB - 1).start()

            gather_descr(s).wait()

            @pl.when(s >= NOB)
            def _():
                write_descr(s - NOB).wait()

            gslot = lax.broadcast(lax.rem(s, NGB), (16,))
            oslot = lax.rem(s, NOB)
            s_splat = lax.broadcast(s, (16,))
            pos_q = [
                plsc.load_gather(pos_v, [q * 16 + iota, s_splat])
                for q in range(EMBED // 16)
            ]
            ob = obuf.at[oslot]
            for d in range(EMBED):
                pq = pos_q[d // 16][d % 16]
                dvec = jnp.full((16,), d, jnp.int32)
                for jv in range(BBLK // 16):
                    vals = plsc.load_gather(gbuf, [gslot, jvecs[jv], dvec])
                    ob[d, pl.ds(jv * 16, 16)] = vals + pq

            write_descr(s).start()
            return carry

        lax.fori_loop(0, SEQ, s_body, 0)
        write_descr(SEQ - 2).wait()
        write_descr(SEQ - 1).wait()

    return body


_sc_kernel = _make_kernel()

_TBLK = 2048


def _prep_body(x_ref, y_ref):
    y_ref[:, 0:EMBED] = x_ref[...].T


def _tc_prep(tab_t):
    grid = (VOCAB + _TBLK - 1) // _TBLK
    return pl.pallas_call(
        _prep_body,
        grid=(grid,),
        in_specs=[pl.BlockSpec((EMBED, _TBLK), lambda j: (0, j))],
        out_specs=pl.BlockSpec((_TBLK, 128), lambda j: (j, 0)),
        out_shape=jax.ShapeDtypeStruct((VOCAB, 128), jnp.float32),
        compiler_params=pltpu.CompilerParams(
            dimension_semantics=("arbitrary",)),
    )(tab_t)


def kernel(input_ids, token_embedding, position_embedding):
    ids_p = input_ids.T.astype(jnp.int32)
    tab3 = _tc_prep(token_embedding.T)
    pos_p = position_embedding.T
    out_p = _sc_kernel(ids_p, tab3, pos_p)
    return out_p.transpose(2, 0, 1)


# speedup vs baseline: 1.4065x; 1.0914x over previous
import functools

import jax
import jax.numpy as jnp
from jax import lax
from jax.experimental import pallas as pl
from jax.experimental.pallas import tpu as pltpu
from jax.experimental.pallas import tpu_sc as plsc

VOCAB = 1000000
EMBED = 64
SEQ = 77
BATCH = 4096

NC = 2
NS = 16
NW = NC * NS

BBLK = BATCH // NW
NGB = 4
NOB = 2


def _make_kernel():
    mesh = plsc.VectorSubcoreMesh(core_axis_name="c", subcore_axis_name="s")

    @functools.partial(
        pl.kernel,
        mesh=mesh,
        out_type=jax.ShapeDtypeStruct((SEQ, EMBED, BATCH), jnp.float32),
        scratch_types=[
            pltpu.VMEM((EMBED, 128), jnp.float32),
            pltpu.VMEM((SEQ, BBLK), jnp.int32),
            pltpu.VMEM((NGB, BBLK, 128), jnp.float32),
            pltpu.VMEM((NOB, EMBED, BBLK), jnp.float32),
            pltpu.SemaphoreType.DMA((NGB,)),
            pltpu.SemaphoreType.DMA((NOB,)),
        ],
        compiler_params=pltpu.CompilerParams(
            use_tc_tiling_on_sc=True, needs_layout_passes=False),
    )
    def body(ids_hbm, tab_hbm, pos_hbm, out_hbm, pos_v, ids_v, gbuf, obuf,
             gsem, wsem):
        wid = lax.axis_index("s") * NC + lax.axis_index("c")
        b0 = wid * BBLK

        pltpu.sync_copy(pos_hbm, pos_v)
        pltpu.sync_copy(ids_hbm.at[:, pl.ds(b0, BBLK)], ids_v)
        iota = lax.iota(jnp.int32, 16)

        def gather_descr(s):
            slot = lax.rem(s, NGB)
            return pltpu.make_async_copy(
                tab_hbm.at[ids_v.at[s]], gbuf.at[slot], gsem.at[slot])

        def write_descr(s):
            slot = lax.rem(s, NOB)
            return pltpu.make_async_copy(
                obuf.at[slot], out_hbm.at[s, :, pl.ds(b0, BBLK)],
                wsem.at[slot])

        for s in range(NGB - 1):
            gather_descr(s).start()

        jvecs = [jv * 16 + iota for jv in range(BBLK // 16)]

        def s_body(s, carry):
            @pl.when(s + NGB - 1 < SEQ)
            def _():
                gather_descr(s + NGB - 1).start()

            gather_descr(s).wait()

            @pl.when(s >= NOB)
            def _():
                write_descr(s - NOB).wait()

            gslot = lax.rem(s, NGB)
            oslot = lax.rem(s, NOB)
            s_splat = lax.broadcast(s, (16,))
            pos_q = [
                plsc.load_gather(pos_v, [q * 16 + iota, s_splat])
                for q in range(EMBED // 16)
            ]
            ob = obuf.at[oslot]
            dvecs = [q * 16 + iota for q in range(EMBED // 16)]

            def j_body(t, carry2):
                for k in range(4):
                    j = t * 4 + k
                    js = lax.broadcast(j, (16,))
                    for q in range(EMBED // 16):
                        v = gbuf[gslot, j, pl.ds(q * 16, 16)] + pos_q[q]
                        plsc.store_scatter(ob, [dvecs[q], js], v)
                return carry2

            lax.fori_loop(0, BBLK // 4, j_body, 0)

            write_descr(s).start()
            return carry

        lax.fori_loop(0, SEQ, s_body, 0)
        write_descr(SEQ - 2).wait()
        write_descr(SEQ - 1).wait()

    return body


_sc_kernel = _make_kernel()

_TBLK = 2048


def _prep_body(x_ref, y_ref):
    x = x_ref[...]
    ii = lax.broadcasted_iota(jnp.int32, (EMBED, EMBED), 0)
    jj = lax.broadcasted_iota(jnp.int32, (EMBED, EMBED), 1)
    eye = (ii == jj).astype(jnp.float32)
    y_ref[:, 0:EMBED] = lax.dot_general(
        x, eye, (((0,), (0,)), ((), ())),
        preferred_element_type=jnp.float32)


def _tc_prep(tab_t):
    grid = (VOCAB + _TBLK - 1) // _TBLK
    return pl.pallas_call(
        _prep_body,
        grid=(grid,),
        in_specs=[pl.BlockSpec((EMBED, _TBLK), lambda j: (0, j))],
        out_specs=pl.BlockSpec((_TBLK, 128), lambda j: (j, 0)),
        out_shape=jax.ShapeDtypeStruct((VOCAB, 128), jnp.float32),
        compiler_params=pltpu.CompilerParams(
            dimension_semantics=("arbitrary",)),
    )(tab_t)


def kernel(input_ids, token_embedding, position_embedding):
    ids_p = input_ids.T.astype(jnp.int32)
    tab3 = _tc_prep(token_embedding.T)
    pos_p = position_embedding.T
    out_p = _sc_kernel(ids_p, tab3, pos_p)
    return out_p.transpose(2, 0, 1)


# speedup vs baseline: 1.8338x; 1.3038x over previous
import functools

import jax
import jax.numpy as jnp
from jax import lax
from jax.experimental import pallas as pl
from jax.experimental.pallas import tpu as pltpu
from jax.experimental.pallas import tpu_sc as plsc

VOCAB = 1000000
EMBED = 64
SEQ = 77
BATCH = 4096

NC = 2
NS = 16
NW = NC * NS

BBLK = BATCH // NW
NGB = 4
NOB = 2


def _make_kernel():
    mesh = plsc.VectorSubcoreMesh(core_axis_name="c", subcore_axis_name="s")

    @functools.partial(
        pl.kernel,
        mesh=mesh,
        out_type=jax.ShapeDtypeStruct((SEQ, EMBED, BATCH), jnp.float32),
        scratch_types=[
            pltpu.VMEM((EMBED, 128), jnp.float32),
            pltpu.VMEM((SEQ, BBLK), jnp.int32),
            pltpu.VMEM((NGB, BBLK, 128), jnp.float32),
            pltpu.VMEM((NOB, EMBED, BBLK), jnp.float32),
            pltpu.SemaphoreType.DMA((NGB,)),
            pltpu.SemaphoreType.DMA((NOB,)),
        ],
        compiler_params=pltpu.CompilerParams(
            use_tc_tiling_on_sc=True, needs_layout_passes=False),
    )
    def body(ids_hbm, tab_hbm, pos_hbm, out_hbm, pos_v, ids_v, gbuf, obuf,
             gsem, wsem):
        wid = lax.axis_index("s") * NC + lax.axis_index("c")
        b0 = wid * BBLK

        pltpu.sync_copy(pos_hbm, pos_v)
        pltpu.sync_copy(ids_hbm.at[:, pl.ds(b0, BBLK)], ids_v)
        iota = lax.iota(jnp.int32, 16)

        def gather_descr(s):
            slot = lax.rem(s, NGB)
            return pltpu.make_async_copy(
                tab_hbm.at[ids_v.at[s]], gbuf.at[slot], gsem.at[slot])

        def write_descr(s):
            slot = lax.rem(s, NOB)
            return pltpu.make_async_copy(
                obuf.at[slot], out_hbm.at[s, :, pl.ds(b0, BBLK)],
                wsem.at[slot])

        for s in range(NGB - 1):
            gather_descr(s).start()

        jvecs = [jv * 16 + iota for jv in range(BBLK // 16)]

        def s_body(s, carry):
            @pl.when(s + NGB - 1 < SEQ)
            def _():
                gather_descr(s + NGB - 1).start()

            gather_descr(s).wait()

            @pl.when(s >= NOB)
            def _():
                write_descr(s - NOB).wait()

            gslot = lax.rem(s, NGB)
            oslot = lax.rem(s, NOB)
            s_splat = lax.broadcast(s, (16,))
            pos_q = [
                plsc.load_gather(pos_v, [q * 16 + iota, s_splat])
                for q in range(EMBED // 16)
            ]
            ob = obuf.at[oslot]
            dvecs = [q * 16 + iota for q in range(EMBED // 16)]

            def j_body(t, carry2):
                for k in range(8):
                    j = t * 8 + k
                    js = lax.broadcast(j, (16,))
                    for q in range(EMBED // 16):
                        v = gbuf[gslot, j, pl.ds(q * 16, 16)] + pos_q[q]
                        plsc.store_scatter(ob, [dvecs[q], js], v)
                return carry2

            lax.fori_loop(0, BBLK // 8, j_body, 0)

            write_descr(s).start()
            return carry

        lax.fori_loop(0, SEQ, s_body, 0)
        write_descr(SEQ - 2).wait()
        write_descr(SEQ - 1).wait()

    return body


_sc_kernel = _make_kernel()

_TBLK = 8192


def _prep_body(x_ref, y_ref):
    x = x_ref[...]
    ii = lax.broadcasted_iota(jnp.int32, (EMBED, EMBED), 0)
    jj = lax.broadcasted_iota(jnp.int32, (EMBED, EMBED), 1)
    eye = (ii == jj).astype(jnp.float32)
    y_ref[:, 0:EMBED] = lax.dot_general(
        x, eye, (((0,), (0,)), ((), ())),
        preferred_element_type=jnp.float32)


def _tc_prep(tab_t):
    grid = (VOCAB + _TBLK - 1) // _TBLK
    return pl.pallas_call(
        _prep_body,
        grid=(grid,),
        in_specs=[pl.BlockSpec((EMBED, _TBLK), lambda j: (0, j))],
        out_specs=pl.BlockSpec((_TBLK, 128), lambda j: (j, 0)),
        out_shape=jax.ShapeDtypeStruct((VOCAB, 128), jnp.float32),
        compiler_params=pltpu.CompilerParams(
            dimension_semantics=("arbitrary",)),
    )(tab_t)


def kernel(input_ids, token_embedding, position_embedding):
    ids_p = input_ids.T.astype(jnp.int32)
    tab3 = _tc_prep(token_embedding.T)
    pos_p = position_embedding.T
    out_p = _sc_kernel(ids_p, tab3, pos_p)
    return out_p.transpose(2, 0, 1)


# speedup vs baseline: 1.9925x; 1.0866x over previous
import functools

import jax
import jax.numpy as jnp
from jax import lax
from jax.experimental import pallas as pl
from jax.experimental.pallas import tpu as pltpu
from jax.experimental.pallas import tpu_sc as plsc

VOCAB = 1000000
EMBED = 64
SEQ = 77
BATCH = 4096

NC = 2
NS = 16
NW = NC * NS

BBLK = BATCH // NW
NGB = 5
NOB = 2


def _make_kernel():
    mesh = plsc.VectorSubcoreMesh(core_axis_name="c", subcore_axis_name="s")

    @functools.partial(
        pl.kernel,
        mesh=mesh,
        out_type=jax.ShapeDtypeStruct((SEQ, EMBED, BATCH), jnp.float32),
        scratch_types=[
            pltpu.VMEM((EMBED, 128), jnp.float32),
            pltpu.VMEM((SEQ, BBLK), jnp.int32),
            pltpu.VMEM((NGB, BBLK, 128), jnp.float32),
            pltpu.VMEM((NOB, EMBED, BBLK), jnp.float32),
            pltpu.SemaphoreType.DMA((NGB,)),
            pltpu.SemaphoreType.DMA((NOB,)),
        ],
        compiler_params=pltpu.CompilerParams(
            use_tc_tiling_on_sc=True, needs_layout_passes=False),
    )
    def body(ids_hbm, tab_hbm, pos_hbm, out_hbm, pos_v, ids_v, gbuf, obuf,
             gsem, wsem):
        wid = lax.axis_index("s") * NC + lax.axis_index("c")
        b0 = wid * BBLK

        pltpu.sync_copy(pos_hbm, pos_v)
        pltpu.sync_copy(ids_hbm.at[:, pl.ds(b0, BBLK)], ids_v)
        iota = lax.iota(jnp.int32, 16)

        def gather_descr(s):
            slot = lax.rem(s, NGB)
            return pltpu.make_async_copy(
                tab_hbm.at[ids_v.at[s]], gbuf.at[slot], gsem.at[slot])

        def write_descr(s):
            slot = lax.rem(s, NOB)
            return pltpu.make_async_copy(
                obuf.at[slot], out_hbm.at[s, :, pl.ds(b0, BBLK)],
                wsem.at[slot])

        for s in range(NGB - 1):
            gather_descr(s).start()

        jvecs = [jv * 16 + iota for jv in range(BBLK // 16)]

        def s_body(s, carry):
            @pl.when(s + NGB - 1 < SEQ)
            def _():
                gather_descr(s + NGB - 1).start()

            gather_descr(s).wait()

            @pl.when(s >= NOB)
            def _():
                write_descr(s - NOB).wait()

            gslot = lax.rem(s, NGB)
            oslot = lax.rem(s, NOB)
            s_splat = lax.broadcast(s, (16,))
            pos_q = [
                plsc.load_gather(pos_v, [q * 16 + iota, s_splat])
                for q in range(EMBED // 16)
            ]
            ob = obuf.at[oslot]
            dvecs = [q * 16 + iota for q in range(EMBED // 16)]

            def j_body(t, carry2):
                NT = 4
                base = t * NT
                js = [lax.broadcast(base + k, (16,)) for k in range(NT)]
                loads = [
                    [gbuf[gslot, base + k, pl.ds(q * 16, 16)]
                     for q in range(EMBED // 16)]
                    for k in range(NT)
                ]
                sums = [
                    [loads[k][q] + pos_q[q] for q in range(EMBED // 16)]
                    for k in range(NT)
                ]
                for k in range(NT):
                    for q in range(EMBED // 16):
                        plsc.store_scatter(ob, [dvecs[q], js[k]], sums[k][q])
                return carry2

            lax.fori_loop(0, BBLK // 4, j_body, 0)

            write_descr(s).start()
            return carry

        lax.fori_loop(0, SEQ, s_body, 0)
        write_descr(SEQ - 2).wait()
        write_descr(SEQ - 1).wait()

    return body


_sc_kernel = _make_kernel()

_TBLK = 8192


def _prep_body(x_ref, y_ref):
    x = x_ref[...]
    ii = lax.broadcasted_iota(jnp.int32, (EMBED, EMBED), 0)
    jj = lax.broadcasted_iota(jnp.int32, (EMBED, EMBED), 1)
    eye = (ii == jj).astype(jnp.float32)
    y_ref[:, 0:EMBED] = lax.dot_general(
        x, eye, (((0,), (0,)), ((), ())),
        preferred_element_type=jnp.float32)


def _tc_prep(tab_t):
    grid = (VOCAB + _TBLK - 1) // _TBLK
    return pl.pallas_call(
        _prep_body,
        grid=(grid,),
        in_specs=[pl.BlockSpec((EMBED, _TBLK), lambda j: (0, j))],
        out_specs=pl.BlockSpec((_TBLK, 128), lambda j: (j, 0)),
        out_shape=jax.ShapeDtypeStruct((VOCAB, 128), jnp.float32),
        compiler_params=pltpu.CompilerParams(
            dimension_semantics=("arbitrary",)),
    )(tab_t)


def kernel(input_ids, token_embedding, position_embedding):
    ids_p = input_ids.T.astype(jnp.int32)
    tab3 = _tc_prep(token_embedding.T)
    pos_p = position_embedding.T
    out_p = _sc_kernel(ids_p, tab3, pos_p)
    return out_p.transpose(2, 0, 1)


# speedup vs baseline: 2.0672x; 1.0375x over previous
import functools

import jax
import jax.numpy as jnp
from jax import lax
from jax.experimental import pallas as pl
from jax.experimental.pallas import tpu as pltpu
from jax.experimental.pallas import tpu_sc as plsc

VOCAB = 1000000
EMBED = 64
SEQ = 77
BATCH = 4096

NC = 2
NS = 16
NW = NC * NS

BBLK = BATCH // NW
NGB = 5
NOB = 2


def _make_kernel():
    mesh = plsc.VectorSubcoreMesh(core_axis_name="c", subcore_axis_name="s")

    @functools.partial(
        pl.kernel,
        mesh=mesh,
        out_type=jax.ShapeDtypeStruct((SEQ, EMBED, BATCH), jnp.float32),
        scratch_types=[
            pltpu.VMEM((EMBED, 128), jnp.float32),
            pltpu.VMEM((SEQ, BBLK), jnp.int32),
            pltpu.VMEM((NGB, BBLK, 128), jnp.float32),
            pltpu.VMEM((NOB, EMBED, BBLK), jnp.float32),
            pltpu.SemaphoreType.DMA((NGB,)),
            pltpu.SemaphoreType.DMA((NOB,)),
        ],
        compiler_params=pltpu.CompilerParams(
            use_tc_tiling_on_sc=True, needs_layout_passes=False),
    )
    def body(ids_hbm, tab_hbm, pos_hbm, out_hbm, pos_v, ids_v, gbuf, obuf,
             gsem, wsem):
        wid = lax.axis_index("s") * NC + lax.axis_index("c")
        b0 = wid * BBLK

        pltpu.sync_copy(pos_hbm, pos_v)
        pltpu.sync_copy(ids_hbm.at[:, pl.ds(b0, BBLK)], ids_v)
        iota = lax.iota(jnp.int32, 16)

        def gather_descr(s):
            slot = lax.rem(s, NGB)
            return pltpu.make_async_copy(
                tab_hbm.at[ids_v.at[s]], gbuf.at[slot], gsem.at[slot])

        def write_descr(s):
            slot = lax.rem(s, NOB)
            return pltpu.make_async_copy(
                obuf.at[slot], out_hbm.at[s, :, pl.ds(b0, BBLK)],
                wsem.at[slot])

        for s in range(NGB - 1):
            gather_descr(s).start()

        jvecs = [jv * 16 + iota for jv in range(BBLK // 16)]

        def s_body(s, carry):
            @pl.when(s + NGB - 1 < SEQ)
            def _():
                gather_descr(s + NGB - 1).start()

            gather_descr(s).wait()

            @pl.when(s >= NOB)
            def _():
                write_descr(s - NOB).wait()

            gslot = lax.rem(s, NGB)
            oslot = lax.rem(s, NOB)
            s_splat = lax.broadcast(s, (16,))
            pos_q = [
                plsc.load_gather(pos_v, [q * 16 + iota, s_splat])
                for q in range(EMBED // 16)
            ]
            ob = obuf.at[oslot]
            dvecs = [q * 16 + iota for q in range(EMBED // 16)]

            def j_body(t, carry2):
                NT = 4
                base = t * NT
                js = [lax.broadcast(base + k, (16,)) for k in range(NT)]
                loads = [
                    [gbuf[gslot, base + k, pl.ds(q * 16, 16)]
                     for q in range(EMBED // 16)]
                    for k in range(NT)
                ]
                sums = [
                    [loads[k][q] + pos_q[q] for q in range(EMBED // 16)]
                    for k in range(NT)
                ]
                for k in range(NT):
                    for q in range(EMBED // 16):
                        plsc.store_scatter(ob, [dvecs[q], js[k]], sums[k][q])
                return carry2

            lax.fori_loop(0, BBLK // 4, j_body, 0)

            write_descr(s).start()
            return carry

        lax.fori_loop(0, SEQ, s_body, 0)
        write_descr(SEQ - 2).wait()
        write_descr(SEQ - 1).wait()

    return body


_sc_kernel = _make_kernel()

_TBLK = 16384


def _prep_body(x_ref, y_ref):
    x = x_ref[...]
    ii = lax.broadcasted_iota(jnp.int32, (EMBED, EMBED), 0)
    jj = lax.broadcasted_iota(jnp.int32, (EMBED, EMBED), 1)
    eye = (ii == jj).astype(jnp.float32)
    y_ref[:, 0:EMBED] = lax.dot_general(
        x, eye, (((0,), (0,)), ((), ())),
        preferred_element_type=jnp.float32)


def _tc_prep(tab_t):
    grid = (VOCAB + _TBLK - 1) // _TBLK
    return pl.pallas_call(
        _prep_body,
        grid=(grid,),
        in_specs=[pl.BlockSpec((EMBED, _TBLK), lambda j: (0, j))],
        out_specs=pl.BlockSpec((_TBLK, 128), lambda j: (j, 0)),
        out_shape=jax.ShapeDtypeStruct((VOCAB, 128), jnp.float32),
        compiler_params=pltpu.CompilerParams(
            dimension_semantics=("arbitrary",)),
    )(tab_t)


def kernel(input_ids, token_embedding, position_embedding):
    ids_p = input_ids.T.astype(jnp.int32)
    tab3 = _tc_prep(token_embedding.T)
    pos_p = position_embedding.T
    out_p = _sc_kernel(ids_p, tab3, pos_p)
    return out_p.transpose(2, 0, 1)


# speedup vs baseline: 2.0883x; 1.0102x over previous
import functools

import jax
import jax.numpy as jnp
from jax import lax
from jax.experimental import pallas as pl
from jax.experimental.pallas import tpu as pltpu
from jax.experimental.pallas import tpu_sc as plsc

VOCAB = 1000000
EMBED = 64
SEQ = 77
BATCH = 4096

NC = 2
NS = 16
NW = NC * NS

BBLK = BATCH // NW
NGB = 5
NOB = 3


def _make_kernel():
    mesh = plsc.VectorSubcoreMesh(core_axis_name="c", subcore_axis_name="s")

    @functools.partial(
        pl.kernel,
        mesh=mesh,
        out_type=jax.ShapeDtypeStruct((SEQ, EMBED, BATCH), jnp.float32),
        scratch_types=[
            pltpu.VMEM((EMBED, 128), jnp.float32),
            pltpu.VMEM((SEQ, BBLK), jnp.int32),
            pltpu.VMEM((NGB, BBLK, 128), jnp.float32),
            pltpu.VMEM((NOB, EMBED, BBLK), jnp.float32),
            pltpu.SemaphoreType.DMA((NGB,)),
            pltpu.SemaphoreType.DMA((NOB,)),
        ],
        compiler_params=pltpu.CompilerParams(
            use_tc_tiling_on_sc=True, needs_layout_passes=False),
    )
    def body(ids_hbm, tab_hbm, pos_hbm, out_hbm, pos_v, ids_v, gbuf, obuf,
             gsem, wsem):
        wid = lax.axis_index("s") * NC + lax.axis_index("c")
        b0 = wid * BBLK

        pltpu.sync_copy(pos_hbm, pos_v)
        pltpu.sync_copy(ids_hbm.at[:, pl.ds(b0, BBLK)], ids_v)
        iota = lax.iota(jnp.int32, 16)

        def gather_descr(s):
            slot = lax.rem(s, NGB)
            return pltpu.make_async_copy(
                tab_hbm.at[ids_v.at[s]], gbuf.at[slot], gsem.at[slot])

        def write_descr(s):
            slot = lax.rem(s, NOB)
            return pltpu.make_async_copy(
                obuf.at[slot], out_hbm.at[s, :, pl.ds(b0, BBLK)],
                wsem.at[slot])

        for s in range(NGB - 1):
            gather_descr(s).start()

        jvecs = [jv * 16 + iota for jv in range(BBLK // 16)]

        def s_body(s, carry):
            @pl.when(s + NGB - 1 < SEQ)
            def _():
                gather_descr(s + NGB - 1).start()

            gather_descr(s).wait()

            @pl.when(s >= NOB)
            def _():
                write_descr(s - NOB).wait()

            gslot = lax.rem(s, NGB)
            oslot = lax.rem(s, NOB)
            s_splat = lax.broadcast(s, (16,))
            pos_q = [
                plsc.load_gather(pos_v, [q * 16 + iota, s_splat])
                for q in range(EMBED // 16)
            ]
            ob = obuf.at[oslot]
            dvecs = [q * 16 + iota for q in range(EMBED // 16)]

            def j_body(t, carry2):
                NT = 8
                base = t * NT
                js = [lax.broadcast(base + k, (16,)) for k in range(NT)]
                loads = [
                    [gbuf[gslot, base + k, pl.ds(q * 16, 16)]
                     for q in range(EMBED // 16)]
                    for k in range(NT)
                ]
                sums = [
                    [loads[k][q] + pos_q[q] for q in range(EMBED // 16)]
                    for k in range(NT)
                ]
                for k in range(NT):
                    for q in range(EMBED // 16):
                        plsc.store_scatter(ob, [dvecs[q], js[k]], sums[k][q])
                return carry2

            lax.fori_loop(0, BBLK // 8, j_body, 0)

            write_descr(s).start()
            return carry

        lax.fori_loop(0, SEQ, s_body, 0)
        write_descr(SEQ - 3).wait()
        write_descr(SEQ - 2).wait()
        write_descr(SEQ - 1).wait()

    return body


_sc_kernel = _make_kernel()

_TBLK = 16384


def _prep_body(x_ref, y_ref):
    x = x_ref[...]
    ii = lax.broadcasted_iota(jnp.int32, (EMBED, EMBED), 0)
    jj = lax.broadcasted_iota(jnp.int32, (EMBED, EMBED), 1)
    eye = (ii == jj).astype(jnp.float32)
    y_ref[:, 0:EMBED] = lax.dot_general(
        x, eye, (((0,), (0,)), ((), ())),
        preferred_element_type=jnp.float32)


def _tc_prep(tab_t):
    grid = (VOCAB + _TBLK - 1) // _TBLK
    return pl.pallas_call(
        _prep_body,
        grid=(grid,),
        in_specs=[pl.BlockSpec((EMBED, _TBLK), lambda j: (0, j))],
        out_specs=pl.BlockSpec((_TBLK, 128), lambda j: (j, 0)),
        out_shape=jax.ShapeDtypeStruct((VOCAB, 128), jnp.float32),
        compiler_params=pltpu.CompilerParams(
            dimension_semantics=("arbitrary",)),
    )(tab_t)


def kernel(input_ids, token_embedding, position_embedding):
    ids_p = input_ids.T.astype(jnp.int32)
    tab3 = _tc_prep(token_embedding.T)
    pos_p = position_embedding.T
    out_p = _sc_kernel(ids_p, tab3, pos_p)
    return out_p.transpose(2, 0, 1)
